# baseline (speedup 1.0000x reference)
import functools

import jax
import jax.numpy as jnp
from jax import lax
from jax.experimental import pallas as pl
from jax.experimental.pallas import tpu as pltpu
from jax.experimental.pallas import tpu_sc as plsc

NUM_EMBEDDINGS = 1024
EMBEDDING_DIM = 64
COMMITMENT_COST = 0.25

N_ROWS = 32 * 576
BLOCK_R = 2304
N_BLOCKS = N_ROWS // BLOCK_R


def _vq_tc_body(x_ref, e_ref, esq_ref, idx_ref, sse_ref, cnt_ref,
                loss_ref, ppl_ref):
    step = pl.program_id(0)
    x = x_ref[...]
    e = e_ref[...]
    sm2 = jax.lax.dot_general(-2.0 * x, e, (((1,), (1,)), ((), ())),
                              preferred_element_type=jnp.float32)
    xsq = jnp.sum(x * x, axis=1, keepdims=True)
    d = (xsq + esq_ref[...]) + sm2

    m = jnp.min(d, axis=1, keepdims=True)
    iota = jax.lax.broadcasted_iota(jnp.int32, d.shape, 1)
    idx = jnp.min(jnp.where(d == m, iota, NUM_EMBEDDINGS),
                  axis=1, keepdims=True)
    idx_ref[...] = idx

    sse_part = jnp.sum(m).reshape(1, 1)
    oh = (iota == idx).astype(jnp.float32)
    ones_row = jnp.ones((1, BLOCK_R), jnp.float32)
    cnt_part = jax.lax.dot_general(ones_row, oh, (((1,), (0,)), ((), ())),
                                   preferred_element_type=jnp.float32)

    @pl.when(step == 0)
    def _init():
        sse_ref[...] = jnp.zeros_like(sse_ref)
        cnt_ref[...] = jnp.zeros_like(cnt_ref)

    sse_ref[...] += sse_part
    cnt_ref[...] += cnt_part

    @pl.when(step == N_BLOCKS - 1)
    def _finalize():
        mean_err = sse_ref[...] / (N_ROWS * EMBEDDING_DIM)
        loss_ref[...] = mean_err + COMMITMENT_COST * mean_err
        p = cnt_ref[...] / N_ROWS
        ent = jnp.sum(p * jnp.log(p + 1e-10)).reshape(1, 1)
        ppl_ref[...] = jnp.exp(-ent)


def _vq_tc_call(flat_x, embedding, esq):
    out_shapes = (
        jax.ShapeDtypeStruct((N_ROWS, 1), jnp.int32),
        jax.ShapeDtypeStruct((1, 1), jnp.float32),
        jax.ShapeDtypeStruct((1, NUM_EMBEDDINGS), jnp.float32),
        jax.ShapeDtypeStruct((1, 1), jnp.float32),
        jax.ShapeDtypeStruct((1, 1), jnp.float32),
    )
    grid = (N_BLOCKS,)
    in_specs = [
        pl.BlockSpec((BLOCK_R, EMBEDDING_DIM), lambda i: (i, 0)),
        pl.BlockSpec((NUM_EMBEDDINGS, EMBEDDING_DIM), lambda i: (0, 0)),
        pl.BlockSpec((1, NUM_EMBEDDINGS), lambda i: (0, 0)),
    ]
    out_specs = (
        pl.BlockSpec((BLOCK_R, 1), lambda i: (i, 0)),
        pl.BlockSpec((1, 1), lambda i: (0, 0)),
        pl.BlockSpec((1, NUM_EMBEDDINGS), lambda i: (0, 0)),
        pl.BlockSpec((1, 1), lambda i: (0, 0)),
        pl.BlockSpec((1, 1), lambda i: (0, 0)),
    )
    return pl.pallas_call(
        _vq_tc_body,
        grid=grid,
        in_specs=in_specs,
        out_specs=out_specs,
        out_shape=out_shapes,
    )(flat_x, embedding, esq)


_SC_INFO = plsc.get_sparse_core_info()
_NC, _NS = _SC_INFO.num_cores, _SC_INFO.num_subcores
_NW = _NC * _NS
_B_PER_W = N_ROWS // _NW


def _sc_gather_body(emb_hbm, idx_hbm, out_hbm, idx_v, rows_v, sem):
    wid = lax.axis_index("s") * _NC + lax.axis_index("c")
    base = wid * _B_PER_W
    pltpu.sync_copy(idx_hbm.at[pl.ds(base, _B_PER_W)], idx_v)
    pltpu.async_copy(emb_hbm.at[idx_v], rows_v, sem).wait()
    pltpu.sync_copy(rows_v, out_hbm.at[pl.ds(base, _B_PER_W)])


def _sc_gather_call(embedding, idx_flat):
    mesh = plsc.VectorSubcoreMesh(core_axis_name="c", subcore_axis_name="s")
    fn = functools.partial(
        pl.kernel, mesh=mesh,
        out_type=jax.ShapeDtypeStruct((N_ROWS, 128), jnp.float32),
        scratch_types=[
            pltpu.VMEM((_B_PER_W,), jnp.int32),
            pltpu.VMEM((_B_PER_W, 128), jnp.float32),
            pltpu.SemaphoreType.DMA,
        ],
    )(_sc_gather_body)
    emb_pad = jnp.pad(embedding, ((0, 0), (0, 128 - EMBEDDING_DIM)))
    return fn(emb_pad, idx_flat)


@jax.jit
def _vq_impl(inputs, embedding):
    input_shape = inputs.shape
    flat_x = inputs.reshape(-1, EMBEDDING_DIM)
    esq = jnp.sum(embedding ** 2, axis=1)[None, :]
    idx, _sse, _cnt, loss, ppl = _vq_tc_call(flat_x, embedding, esq)
    idx_flat = idx.reshape(-1)
    qst = _sc_gather_call(embedding, idx_flat)[:, :EMBEDDING_DIM]
    return (loss.reshape(()), qst.reshape(input_shape), ppl.reshape(()),
            idx_flat.reshape(input_shape[:-1]))


def kernel(inputs, embedding):
    return _vq_impl(inputs, embedding)


# speedup vs baseline: 1.2125x; 1.2125x over previous
import functools

import jax
import jax.numpy as jnp
from jax import lax
from jax.experimental import pallas as pl
from jax.experimental.pallas import tpu as pltpu
from jax.experimental.pallas import tpu_sc as plsc

NUM_EMBEDDINGS = 1024
EMBEDDING_DIM = 64
COMMITMENT_COST = 0.25

N_ROWS = 32 * 576
BLOCK_R = 2304
N_BLOCKS = N_ROWS // BLOCK_R


def _vq_tc_body(x_ref, e_ref, idx_ref, qst_ref, sse_ref, cnt_ref,
                loss_ref, ppl_ref):
    step = pl.program_id(0)
    x = x_ref[...]
    e = e_ref[...]
    esq = jnp.sum(e * e, axis=1, keepdims=True).reshape(1, NUM_EMBEDDINGS)
    sm2 = jax.lax.dot_general(-2.0 * x, e, (((1,), (1,)), ((), ())),
                              preferred_element_type=jnp.float32)
    xsq = jnp.sum(x * x, axis=1, keepdims=True)
    d = (xsq + esq) + sm2

    m = jnp.min(d, axis=1, keepdims=True)
    iota = jax.lax.broadcasted_iota(jnp.int32, d.shape, 1)
    idx = jnp.min(jnp.where(d == m, iota, NUM_EMBEDDINGS),
                  axis=1, keepdims=True)
    idx_ref[...] = idx

    sse_part = jnp.sum(m).reshape(1, 1)
    oh = (iota == idx).astype(jnp.float32)
    q = jax.lax.dot_general(oh, e, (((1,), (0,)), ((), ())),
                            preferred_element_type=jnp.float32)
    qst_ref[...] = q
    ones_row = jnp.ones((1, BLOCK_R), jnp.float32)
    cnt_part = jax.lax.dot_general(ones_row, oh, (((1,), (0,)), ((), ())),
                                   preferred_element_type=jnp.float32)

    @pl.when(step == 0)
    def _init():
        sse_ref[...] = jnp.zeros_like(sse_ref)
        cnt_ref[...] = jnp.zeros_like(cnt_ref)

    sse_ref[...] += sse_part
    cnt_ref[...] += cnt_part

    @pl.when(step == N_BLOCKS - 1)
    def _finalize():
        mean_err = sse_ref[...] / (N_ROWS * EMBEDDING_DIM)
        loss_ref[...] = mean_err + COMMITMENT_COST * mean_err
        p = cnt_ref[...] / N_ROWS
        ent = jnp.sum(p * jnp.log(p + 1e-10)).reshape(1, 1)
        ppl_ref[...] = jnp.exp(-ent)


def _vq_tc_call(flat_x, embedding):
    out_shapes = (
        jax.ShapeDtypeStruct((N_ROWS, 1), jnp.int32),
        jax.ShapeDtypeStruct((N_ROWS, EMBEDDING_DIM), jnp.float32),
        jax.ShapeDtypeStruct((1, 1), jnp.float32),
        jax.ShapeDtypeStruct((1, NUM_EMBEDDINGS), jnp.float32),
        jax.ShapeDtypeStruct((1, 1), jnp.float32),
        jax.ShapeDtypeStruct((1, 1), jnp.float32),
    )
    grid = (N_BLOCKS,)
    in_specs = [
        pl.BlockSpec((BLOCK_R, EMBEDDING_DIM), lambda i: (i, 0)),
        pl.BlockSpec((NUM_EMBEDDINGS, EMBEDDING_DIM), lambda i: (0, 0)),
    ]
    out_specs = (
        pl.BlockSpec((BLOCK_R, 1), lambda i: (i, 0)),
        pl.BlockSpec((BLOCK_R, EMBEDDING_DIM), lambda i: (i, 0)),
        pl.BlockSpec((1, 1), lambda i: (0, 0)),
        pl.BlockSpec((1, NUM_EMBEDDINGS), lambda i: (0, 0)),
        pl.BlockSpec((1, 1), lambda i: (0, 0)),
        pl.BlockSpec((1, 1), lambda i: (0, 0)),
    )
    return pl.pallas_call(
        _vq_tc_body,
        grid=grid,
        in_specs=in_specs,
        out_specs=out_specs,
        out_shape=out_shapes,
    )(flat_x, embedding)


_SC_INFO = plsc.get_sparse_core_info()
_NC, _NS = _SC_INFO.num_cores, _SC_INFO.num_subcores
_NW = _NC * _NS
_B_PER_W = N_ROWS // _NW


def _sc_gather_body(emb_hbm, idx_hbm, out_hbm, idx_v, rows_v, sem):
    wid = lax.axis_index("s") * _NC + lax.axis_index("c")
    base = wid * _B_PER_W
    pltpu.sync_copy(idx_hbm.at[pl.ds(base, _B_PER_W)], idx_v)
    pltpu.async_copy(emb_hbm.at[idx_v], rows_v, sem).wait()
    pltpu.sync_copy(rows_v, out_hbm.at[pl.ds(base, _B_PER_W)])


def _sc_gather_call(embedding, idx_flat):
    mesh = plsc.VectorSubcoreMesh(core_axis_name="c", subcore_axis_name="s")
    fn = functools.partial(
        pl.kernel, mesh=mesh,
        out_type=jax.ShapeDtypeStruct((N_ROWS, 128), jnp.float32),
        scratch_types=[
            pltpu.VMEM((_B_PER_W,), jnp.int32),
            pltpu.VMEM((_B_PER_W, 128), jnp.float32),
            pltpu.SemaphoreType.DMA,
        ],
    )(_sc_gather_body)
    emb_pad = jnp.pad(embedding, ((0, 0), (0, 128 - EMBEDDING_DIM)))
    return fn(emb_pad, idx_flat)


@jax.jit
def _vq_impl(inputs, embedding):
    input_shape = inputs.shape
    flat_x = inputs.reshape(-1, EMBEDDING_DIM)
    idx, qst, _sse, _cnt, loss, ppl = _vq_tc_call(flat_x, embedding)
    return (loss.reshape(()), qst.reshape(input_shape), ppl.reshape(()),
            idx.reshape(input_shape[:-1]))


def kernel(inputs, embedding):
    return _vq_impl(inputs, embedding)


# speedup vs baseline: 1.2556x; 1.0355x over previous
import functools

import jax
import jax.numpy as jnp
from jax import lax
from jax.experimental import pallas as pl
from jax.experimental.pallas import tpu as pltpu
from jax.experimental.pallas import tpu_sc as plsc

NUM_EMBEDDINGS = 1024
EMBEDDING_DIM = 64
COMMITMENT_COST = 0.25

N_ROWS = 32 * 576
BLOCK_R = 2304
N_BLOCKS = N_ROWS // BLOCK_R


def _vq_tc_body(x_ref, e_ref, idx_ref, qst_ref, sse_ref, cnt_ref,
                loss_ref, ppl_ref):
    step = pl.program_id(0)
    x = x_ref[...]
    e = e_ref[...]
    esq = jnp.sum(e * e, axis=1, keepdims=True).reshape(1, NUM_EMBEDDINGS)
    sm2 = jax.lax.dot_general(-2.0 * x, e, (((1,), (1,)), ((), ())),
                              preferred_element_type=jnp.float32)
    xsq = jnp.sum(x * x, axis=1, keepdims=True)
    d = (xsq + esq) + sm2

    m = jnp.min(d, axis=1, keepdims=True)
    iota_f = jax.lax.broadcasted_iota(jnp.int32, d.shape, 1).astype(jnp.float32)
    c = jnp.where(d == m, iota_f, jnp.float32(NUM_EMBEDDINGS))
    idx_f = jnp.min(c, axis=1, keepdims=True)
    idx_ref[...] = idx_f.astype(jnp.int32)

    sse_part = jnp.sum(m).reshape(1, 1)
    oh = (c == idx_f).astype(jnp.float32)
    q = jax.lax.dot_general(oh, e, (((1,), (0,)), ((), ())),
                            preferred_element_type=jnp.float32)
    qst_ref[...] = q
    ones_row = jnp.ones((1, BLOCK_R), jnp.float32)
    cnt_part = jax.lax.dot_general(ones_row, oh, (((1,), (0,)), ((), ())),
                                   preferred_element_type=jnp.float32)

    @pl.when(step == 0)
    def _init():
        sse_ref[...] = jnp.zeros_like(sse_ref)
        cnt_ref[...] = jnp.zeros_like(cnt_ref)

    sse_ref[...] += sse_part
    cnt_ref[...] += cnt_part

    @pl.when(step == N_BLOCKS - 1)
    def _finalize():
        mean_err = sse_ref[...] / (N_ROWS * EMBEDDING_DIM)
        loss_ref[...] = mean_err + COMMITMENT_COST * mean_err
        p = cnt_ref[...] / N_ROWS
        ent = jnp.sum(p * jnp.log(p + 1e-10)).reshape(1, 1)
        ppl_ref[...] = jnp.exp(-ent)


def _vq_tc_call(flat_x, embedding):
    out_shapes = (
        jax.ShapeDtypeStruct((N_ROWS, 1), jnp.int32),
        jax.ShapeDtypeStruct((N_ROWS, EMBEDDING_DIM), jnp.float32),
        jax.ShapeDtypeStruct((1, 1), jnp.float32),
        jax.ShapeDtypeStruct((1, NUM_EMBEDDINGS), jnp.float32),
        jax.ShapeDtypeStruct((1, 1), jnp.float32),
        jax.ShapeDtypeStruct((1, 1), jnp.float32),
    )
    grid = (N_BLOCKS,)
    in_specs = [
        pl.BlockSpec((BLOCK_R, EMBEDDING_DIM), lambda i: (i, 0)),
        pl.BlockSpec((NUM_EMBEDDINGS, EMBEDDING_DIM), lambda i: (0, 0)),
    ]
    out_specs = (
        pl.BlockSpec((BLOCK_R, 1), lambda i: (i, 0)),
        pl.BlockSpec((BLOCK_R, EMBEDDING_DIM), lambda i: (i, 0)),
        pl.BlockSpec((1, 1), lambda i: (0, 0)),
        pl.BlockSpec((1, NUM_EMBEDDINGS), lambda i: (0, 0)),
        pl.BlockSpec((1, 1), lambda i: (0, 0)),
        pl.BlockSpec((1, 1), lambda i: (0, 0)),
    )
    return pl.pallas_call(
        _vq_tc_body,
        grid=grid,
        in_specs=in_specs,
        out_specs=out_specs,
        out_shape=out_shapes,
    )(flat_x, embedding)


_SC_INFO = plsc.get_sparse_core_info()
_NC, _NS = _SC_INFO.num_cores, _SC_INFO.num_subcores
_NW = _NC * _NS
_B_PER_W = N_ROWS // _NW


def _sc_gather_body(emb_hbm, idx_hbm, out_hbm, idx_v, rows_v, sem):
    wid = lax.axis_index("s") * _NC + lax.axis_index("c")
    base = wid * _B_PER_W
    pltpu.sync_copy(idx_hbm.at[pl.ds(base, _B_PER_W)], idx_v)
    pltpu.async_copy(emb_hbm.at[idx_v], rows_v, sem).wait()
    pltpu.sync_copy(rows_v, out_hbm.at[pl.ds(base, _B_PER_W)])


def _sc_gather_call(embedding, idx_flat):
    mesh = plsc.VectorSubcoreMesh(core_axis_name="c", subcore_axis_name="s")
    fn = functools.partial(
        pl.kernel, mesh=mesh,
        out_type=jax.ShapeDtypeStruct((N_ROWS, 128), jnp.float32),
        scratch_types=[
            pltpu.VMEM((_B_PER_W,), jnp.int32),
            pltpu.VMEM((_B_PER_W, 128), jnp.float32),
            pltpu.SemaphoreType.DMA,
        ],
    )(_sc_gather_body)
    emb_pad = jnp.pad(embedding, ((0, 0), (0, 128 - EMBEDDING_DIM)))
    return fn(emb_pad, idx_flat)


@jax.jit
def _vq_impl(inputs, embedding):
    input_shape = inputs.shape
    flat_x = inputs.reshape(-1, EMBEDDING_DIM)
    idx, qst, _sse, _cnt, loss, ppl = _vq_tc_call(flat_x, embedding)
    return (loss.reshape(()), qst.reshape(input_shape), ppl.reshape(()),
            idx.reshape(input_shape[:-1]))


def kernel(inputs, embedding):
    return _vq_impl(inputs, embedding)


# speedup vs baseline: 1.3931x; 1.1096x over previous
import functools

import jax
import jax.numpy as jnp
from jax import lax
from jax.experimental import pallas as pl
from jax.experimental.pallas import tpu as pltpu
from jax.experimental.pallas import tpu_sc as plsc

NUM_EMBEDDINGS = 1024
EMBEDDING_DIM = 64
COMMITMENT_COST = 0.25

N_ROWS = 32 * 576
BLOCK_R = 4608
N_BLOCKS = N_ROWS // BLOCK_R


def _vq_tc_body(x_ref, e_ref, idx_ref, qst_ref, sse_ref, cnt_ref,
                loss_ref, ppl_ref):
    step = pl.program_id(0)
    x = x_ref[...]
    e = e_ref[...]
    esq = jnp.sum(e * e, axis=1, keepdims=True).reshape(1, NUM_EMBEDDINGS)
    sm2 = jax.lax.dot_general(-2.0 * x, e, (((1,), (1,)), ((), ())),
                              preferred_element_type=jnp.float32)
    xsq = jnp.sum(x * x, axis=1, keepdims=True)
    d = (xsq + esq) + sm2

    m = jnp.min(d, axis=1, keepdims=True)
    iota_f = jax.lax.broadcasted_iota(jnp.int32, d.shape, 1).astype(jnp.float32)
    c = jnp.where(d == m, iota_f, jnp.float32(NUM_EMBEDDINGS))
    idx_f = jnp.min(c, axis=1, keepdims=True)
    idx_ref[...] = idx_f.astype(jnp.int32).reshape(BLOCK_R // 576, 576)

    sse_part = jnp.sum(m).reshape(1, 1)
    oh = (c == idx_f).astype(jnp.float32)
    q = jax.lax.dot_general(oh, e, (((1,), (0,)), ((), ())),
                            preferred_element_type=jnp.float32)
    qst_ref[...] = q.reshape(BLOCK_R // 576, 576, EMBEDDING_DIM)
    ones_row = jnp.ones((1, BLOCK_R), jnp.float32)
    cnt_part = jax.lax.dot_general(ones_row, oh, (((1,), (0,)), ((), ())),
                                   preferred_element_type=jnp.float32)

    @pl.when(step == 0)
    def _init():
        sse_ref[...] = jnp.zeros_like(sse_ref)
        cnt_ref[...] = jnp.zeros_like(cnt_ref)

    sse_ref[...] += sse_part
    cnt_ref[...] += cnt_part

    @pl.when(step == N_BLOCKS - 1)
    def _finalize():
        mean_err = sse_ref[...] / (N_ROWS * EMBEDDING_DIM)
        loss_ref[...] = mean_err + COMMITMENT_COST * mean_err
        p = cnt_ref[...] / N_ROWS
        ent = jnp.sum(p * jnp.log(p + 1e-10)).reshape(1, 1)
        ppl_ref[...] = jnp.exp(-ent)


def _vq_tc_call(flat_x, embedding):
    out_shapes = (
        jax.ShapeDtypeStruct((32, 576), jnp.int32),
        jax.ShapeDtypeStruct((32, 576, EMBEDDING_DIM), jnp.float32),
        jax.ShapeDtypeStruct((1, 1), jnp.float32),
        jax.ShapeDtypeStruct((1, NUM_EMBEDDINGS), jnp.float32),
        jax.ShapeDtypeStruct((1, 1), jnp.float32),
        jax.ShapeDtypeStruct((1, 1), jnp.float32),
    )
    grid = (N_BLOCKS,)
    in_specs = [
        pl.BlockSpec((BLOCK_R, EMBEDDING_DIM), lambda i: (i, 0)),
        pl.BlockSpec((NUM_EMBEDDINGS, EMBEDDING_DIM), lambda i: (0, 0)),
    ]
    out_specs = (
        pl.BlockSpec((BLOCK_R // 576, 576), lambda i: (i, 0)),
        pl.BlockSpec((BLOCK_R // 576, 576, EMBEDDING_DIM), lambda i: (i, 0, 0)),
        pl.BlockSpec((1, 1), lambda i: (0, 0)),
        pl.BlockSpec((1, NUM_EMBEDDINGS), lambda i: (0, 0)),
        pl.BlockSpec((1, 1), lambda i: (0, 0)),
        pl.BlockSpec((1, 1), lambda i: (0, 0)),
    )
    return pl.pallas_call(
        _vq_tc_body,
        grid=grid,
        in_specs=in_specs,
        out_specs=out_specs,
        out_shape=out_shapes,
    )(flat_x, embedding)


_SC_INFO = plsc.get_sparse_core_info()
_NC, _NS = _SC_INFO.num_cores, _SC_INFO.num_subcores
_NW = _NC * _NS
_B_PER_W = N_ROWS // _NW


def _sc_gather_body(emb_hbm, idx_hbm, out_hbm, idx_v, rows_v, sem):
    wid = lax.axis_index("s") * _NC + lax.axis_index("c")
    base = wid * _B_PER_W
    pltpu.sync_copy(idx_hbm.at[pl.ds(base, _B_PER_W)], idx_v)
    pltpu.async_copy(emb_hbm.at[idx_v], rows_v, sem).wait()
    pltpu.sync_copy(rows_v, out_hbm.at[pl.ds(base, _B_PER_W)])


def _sc_gather_call(embedding, idx_flat):
    mesh = plsc.VectorSubcoreMesh(core_axis_name="c", subcore_axis_name="s")
    fn = functools.partial(
        pl.kernel, mesh=mesh,
        out_type=jax.ShapeDtypeStruct((N_ROWS, 128), jnp.float32),
        scratch_types=[
            pltpu.VMEM((_B_PER_W,), jnp.int32),
            pltpu.VMEM((_B_PER_W, 128), jnp.float32),
            pltpu.SemaphoreType.DMA,
        ],
    )(_sc_gather_body)
    emb_pad = jnp.pad(embedding, ((0, 0), (0, 128 - EMBEDDING_DIM)))
    return fn(emb_pad, idx_flat)


@jax.jit
def _vq_impl(inputs, embedding):
    flat_x = inputs.reshape(-1, EMBEDDING_DIM)
    idx, qst, _sse, _cnt, loss, ppl = _vq_tc_call(flat_x, embedding)
    return (loss.reshape(()), qst, ppl.reshape(()), idx)


def kernel(inputs, embedding):
    return _vq_impl(inputs, embedding)


# speedup vs baseline: 1.3958x; 1.0019x over previous
import functools

import jax
import jax.numpy as jnp
from jax import lax
from jax.experimental import pallas as pl
from jax.experimental.pallas import tpu as pltpu
from jax.experimental.pallas import tpu_sc as plsc

NUM_EMBEDDINGS = 1024
EMBEDDING_DIM = 64
COMMITMENT_COST = 0.25

N_ROWS = 32 * 576
BLOCK_R = 4608
N_BLOCKS = N_ROWS // BLOCK_R


def _vq_tc_body(x_ref, e_ref, idx_ref, qst_ref, sse_ref, cnt_ref,
                loss_ref, ppl_ref):
    step = pl.program_id(0)
    x = x_ref[...].reshape(BLOCK_R, EMBEDDING_DIM)
    e = e_ref[...]
    esq = jnp.sum(e * e, axis=1, keepdims=True).reshape(1, NUM_EMBEDDINGS)
    sm2 = jax.lax.dot_general(-2.0 * x, e, (((1,), (1,)), ((), ())),
                              preferred_element_type=jnp.float32)
    xsq = jnp.sum(x * x, axis=1, keepdims=True)
    d = (xsq + esq) + sm2

    m = jnp.min(d, axis=1, keepdims=True)
    iota_f = jax.lax.broadcasted_iota(jnp.int32, d.shape, 1).astype(jnp.float32)
    c = jnp.where(d == m, iota_f, jnp.float32(NUM_EMBEDDINGS))
    idx_f = jnp.min(c, axis=1, keepdims=True)
    idx_ref[...] = idx_f.astype(jnp.int32).reshape(BLOCK_R // 576, 576)

    sse_part = jnp.sum(m).reshape(1, 1)
    oh = (c == idx_f).astype(jnp.float32)
    q = jax.lax.dot_general(oh, e, (((1,), (0,)), ((), ())),
                            preferred_element_type=jnp.float32)
    qst_ref[...] = q.reshape(BLOCK_R // 576, 576, EMBEDDING_DIM)
    ones_row = jnp.ones((1, BLOCK_R), jnp.float32)
    cnt_part = jax.lax.dot_general(ones_row, oh, (((1,), (0,)), ((), ())),
                                   preferred_element_type=jnp.float32)

    @pl.when(step == 0)
    def _init():
        sse_ref[...] = jnp.zeros_like(sse_ref)
        cnt_ref[...] = jnp.zeros_like(cnt_ref)

    sse_ref[...] += sse_part
    cnt_ref[...] += cnt_part

    @pl.when(step == N_BLOCKS - 1)
    def _finalize():
        mean_err = sse_ref[...] / (N_ROWS * EMBEDDING_DIM)
        loss_ref[...] = mean_err + COMMITMENT_COST * mean_err
        p = cnt_ref[...] / N_ROWS
        ent = jnp.sum(p * jnp.log(p + 1e-10)).reshape(1, 1)
        ppl_ref[...] = jnp.exp(-ent)


def _vq_tc_call(inputs3d, embedding):
    out_shapes = (
        jax.ShapeDtypeStruct((32, 576), jnp.int32),
        jax.ShapeDtypeStruct((32, 576, EMBEDDING_DIM), jnp.float32),
        jax.ShapeDtypeStruct((1, 1), jnp.float32),
        jax.ShapeDtypeStruct((1, NUM_EMBEDDINGS), jnp.float32),
        jax.ShapeDtypeStruct((1, 1), jnp.float32),
        jax.ShapeDtypeStruct((1, 1), jnp.float32),
    )
    grid = (N_BLOCKS,)
    in_specs = [
        pl.BlockSpec((BLOCK_R // 576, 576, EMBEDDING_DIM), lambda i: (i, 0, 0)),
        pl.BlockSpec((NUM_EMBEDDINGS, EMBEDDING_DIM), lambda i: (0, 0)),
    ]
    out_specs = (
        pl.BlockSpec((BLOCK_R // 576, 576), lambda i: (i, 0)),
        pl.BlockSpec((BLOCK_R // 576, 576, EMBEDDING_DIM), lambda i: (i, 0, 0)),
        pl.BlockSpec((1, 1), lambda i: (0, 0)),
        pl.BlockSpec((1, NUM_EMBEDDINGS), lambda i: (0, 0)),
        pl.BlockSpec((1, 1), lambda i: (0, 0)),
        pl.BlockSpec((1, 1), lambda i: (0, 0)),
    )
    return pl.pallas_call(
        _vq_tc_body,
        grid=grid,
        in_specs=in_specs,
        out_specs=out_specs,
        out_shape=out_shapes,
    )(inputs3d, embedding)


_SC_INFO = plsc.get_sparse_core_info()
_NC, _NS = _SC_INFO.num_cores, _SC_INFO.num_subcores
_NW = _NC * _NS
_B_PER_W = N_ROWS // _NW


def _sc_gather_body(emb_hbm, idx_hbm, out_hbm, idx_v, rows_v, sem):
    wid = lax.axis_index("s") * _NC + lax.axis_index("c")
    base = wid * _B_PER_W
    pltpu.sync_copy(idx_hbm.at[pl.ds(base, _B_PER_W)], idx_v)
    pltpu.async_copy(emb_hbm.at[idx_v], rows_v, sem).wait()
    pltpu.sync_copy(rows_v, out_hbm.at[pl.ds(base, _B_PER_W)])


def _sc_gather_call(embedding, idx_flat):
    mesh = plsc.VectorSubcoreMesh(core_axis_name="c", subcore_axis_name="s")
    fn = functools.partial(
        pl.kernel, mesh=mesh,
        out_type=jax.ShapeDtypeStruct((N_ROWS, 128), jnp.float32),
        scratch_types=[
            pltpu.VMEM((_B_PER_W,), jnp.int32),
            pltpu.VMEM((_B_PER_W, 128), jnp.float32),
            pltpu.SemaphoreType.DMA,
        ],
    )(_sc_gather_body)
    emb_pad = jnp.pad(embedding, ((0, 0), (0, 128 - EMBEDDING_DIM)))
    return fn(emb_pad, idx_flat)


@jax.jit
def _vq_impl(inputs, embedding):
    idx, qst, _sse, _cnt, loss, ppl = _vq_tc_call(inputs, embedding)
    return (loss.reshape(()), qst, ppl.reshape(()), idx)


def kernel(inputs, embedding):
    return _vq_impl(inputs, embedding)


# speedup vs baseline: 1.3963x; 1.0003x over previous
import jax
import jax.numpy as jnp
from jax.experimental import pallas as pl

NUM_EMBEDDINGS = 1024
EMBEDDING_DIM = 64
COMMITMENT_COST = 0.25

N_ROWS = 32 * 576
BLOCK_R = 4608
N_BLOCKS = N_ROWS // BLOCK_R
_ROWS_OUT = BLOCK_R // 576


def _vq_body(x_ref, e_ref, idx_ref, qst_ref, sse_ref, cnt_ref,
             loss_ref, ppl_ref):
    step = pl.program_id(0)
    x = x_ref[...].reshape(BLOCK_R, EMBEDDING_DIM)
    e = e_ref[...]
    esq = jnp.sum(e * e, axis=1, keepdims=True).reshape(1, NUM_EMBEDDINGS)
    sm2 = jax.lax.dot_general(-2.0 * x, e, (((1,), (1,)), ((), ())),
                              preferred_element_type=jnp.float32)
    xsq = jnp.sum(x * x, axis=1, keepdims=True)
    d = (xsq + esq) + sm2

    m = jnp.min(d, axis=1, keepdims=True)
    iota_f = jax.lax.broadcasted_iota(jnp.int32, d.shape, 1).astype(jnp.float32)
    c = jnp.where(d == m, iota_f, jnp.float32(NUM_EMBEDDINGS))
    idx_f = jnp.min(c, axis=1, keepdims=True)
    idx_ref[...] = idx_f.astype(jnp.int32).reshape(_ROWS_OUT, 576)

    sse_part = jnp.sum(m).reshape(1, 1)
    oh = (c == idx_f).astype(jnp.float32)
    q = jax.lax.dot_general(oh, e, (((1,), (0,)), ((), ())),
                            preferred_element_type=jnp.float32)
    qst_ref[...] = q.reshape(_ROWS_OUT, 576, EMBEDDING_DIM)
    ones_row = jnp.ones((1, BLOCK_R), jnp.float32)
    cnt_part = jax.lax.dot_general(ones_row, oh, (((1,), (0,)), ((), ())),
                                   preferred_element_type=jnp.float32)

    @pl.when(step == 0)
    def _init():
        sse_ref[...] = jnp.zeros_like(sse_ref)
        cnt_ref[...] = jnp.zeros_like(cnt_ref)

    sse_ref[...] += sse_part
    cnt_ref[...] += cnt_part

    @pl.when(step == N_BLOCKS - 1)
    def _finalize():
        mean_err = sse_ref[...] / (N_ROWS * EMBEDDING_DIM)
        loss_ref[...] = mean_err + COMMITMENT_COST * mean_err
        p = cnt_ref[...] / N_ROWS
        ent = jnp.sum(p * jnp.log(p + 1e-10)).reshape(1, 1)
        ppl_ref[...] = jnp.exp(-ent)


def _vq_call(inputs3d, embedding):
    out_shapes = (
        jax.ShapeDtypeStruct((32, 576), jnp.int32),
        jax.ShapeDtypeStruct((32, 576, EMBEDDING_DIM), jnp.float32),
        jax.ShapeDtypeStruct((1, 1), jnp.float32),
        jax.ShapeDtypeStruct((1, NUM_EMBEDDINGS), jnp.float32),
        jax.ShapeDtypeStruct((1, 1), jnp.float32),
        jax.ShapeDtypeStruct((1, 1), jnp.float32),
    )
    grid = (N_BLOCKS,)
    in_specs = [
        pl.BlockSpec((_ROWS_OUT, 576, EMBEDDING_DIM), lambda i: (i, 0, 0)),
        pl.BlockSpec((NUM_EMBEDDINGS, EMBEDDING_DIM), lambda i: (0, 0)),
    ]
    out_specs = (
        pl.BlockSpec((_ROWS_OUT, 576), lambda i: (i, 0)),
        pl.BlockSpec((_ROWS_OUT, 576, EMBEDDING_DIM), lambda i: (i, 0, 0)),
        pl.BlockSpec((1, 1), lambda i: (0, 0)),
        pl.BlockSpec((1, NUM_EMBEDDINGS), lambda i: (0, 0)),
        pl.BlockSpec((1, 1), lambda i: (0, 0)),
        pl.BlockSpec((1, 1), lambda i: (0, 0)),
    )
    return pl.pallas_call(
        _vq_body,
        grid=grid,
        in_specs=in_specs,
        out_specs=out_specs,
        out_shape=out_shapes,
    )(inputs3d, embedding)


@jax.jit
def _vq_impl(inputs, embedding):
    idx, qst, _sse, _cnt, loss, ppl = _vq_call(inputs, embedding)
    return (loss.reshape(()), qst, ppl.reshape(()), idx)


def kernel(inputs, embedding):
    return _vq_impl(inputs, embedding)
